# Initial kernel scaffold; baseline (speedup 1.0000x reference)
import functools

import jax
import jax.numpy as jnp
from jax import lax
from jax.experimental import pallas as pl
from jax.experimental.pallas import tpu as pltpu
from jax.experimental.pallas import tpu_sc as plsc

N_NODES = 10000
N_EDGES = 320000
D = 128

NC, NS, LANES = 2, 16, 16
NW = NC * NS
EPW = N_EDGES // NW
CHUNK = 80
NCHUNK = EPW // CHUNK
ZROWS = 4 * CHUNK
N_PAD = 10240
RPT = N_PAD // NS


def _agg_body(with_counts, *refs):
    if with_counts:
        (feat, src3, dst3, part_out, cnt_out,
         idx_s, idx_d, rows, ones, zv, acc, cnt) = refs
    else:
        (feat, src3, dst3, part_out,
         idx_s, idx_d, rows, acc) = refs

    cid = lax.axis_index("c")
    sid = lax.axis_index("s")
    wid = cid * NS + sid

    pltpu.sync_copy(src3.at[wid], idx_s)
    pltpu.sync_copy(dst3.at[wid], idx_d)

    def zrow(i, _):
        for j in range(D // LANES):
            rows[i, pl.ds(j * LANES, LANES)] = jnp.zeros((LANES,), jnp.float32)
        return 0
    lax.fori_loop(0, ZROWS, zrow, 0)
    base = sid * RPT
    pltpu.sync_copy(rows.at[pl.ds(0, ZROWS)], acc.at[pl.ds(base, ZROWS)])
    pltpu.sync_copy(rows.at[pl.ds(0, ZROWS)], acc.at[pl.ds(base + ZROWS, ZROWS)])

    if with_counts:
        def zvec(i, _):
            zv[pl.ds(i * LANES, LANES)] = jnp.zeros((LANES,), jnp.float32)
            return 0
        lax.fori_loop(0, RPT // LANES, zvec, 0)
        pltpu.sync_copy(zv, cnt.at[pl.ds(base, RPT)])
        for j in range(CHUNK // LANES):
            ones[pl.ds(j * LANES, LANES)] = jnp.ones((LANES,), jnp.float32)

    plsc.subcore_barrier()

    def body(g, _):
        buf = rows.at[pl.ds(0, CHUNK)]
        pltpu.sync_copy(feat.at[idx_s.at[g]], buf)
        pltpu.sync_copy(buf, acc.at[idx_d.at[g]], add=True)
        if with_counts:
            pltpu.sync_copy(ones, cnt.at[idx_d.at[g]], add=True)
        return 0
    lax.fori_loop(0, NCHUNK, body, 0)

    plsc.subcore_barrier()

    pltpu.sync_copy(acc.at[pl.ds(base, RPT)], part_out.at[cid, pl.ds(base, RPT)])
    if with_counts:
        pltpu.sync_copy(cnt.at[pl.ds(base, RPT)], cnt_out.at[cid, pl.ds(base, RPT)])


def _make_agg(with_counts):
    out_type = [jax.ShapeDtypeStruct((NC, N_PAD, D), jnp.float32)]
    scratch = [
        pltpu.VMEM((NCHUNK, CHUNK), jnp.int32),
        pltpu.VMEM((NCHUNK, CHUNK), jnp.int32),
        pltpu.VMEM((ZROWS, D), jnp.float32),
    ]
    if with_counts:
        out_type.append(jax.ShapeDtypeStruct((NC, N_PAD), jnp.float32))
        scratch += [
            pltpu.VMEM((CHUNK,), jnp.float32),
            pltpu.VMEM((RPT,), jnp.float32),
        ]
    scratch.append(pltpu.VMEM_SHARED((N_PAD, D), jnp.float32))
    if with_counts:
        scratch.append(pltpu.VMEM_SHARED((N_PAD,), jnp.float32))
    mesh = plsc.VectorSubcoreMesh(
        core_axis_name="c", subcore_axis_name="s",
        num_cores=NC, num_subcores=NS)
    return pl.kernel(
        functools.partial(_agg_body, with_counts),
        out_type=out_type, mesh=mesh, scratch_types=scratch)


_agg_with_counts = _make_agg(True)
_agg_no_counts = _make_agg(False)


def _dense_body(relu, p0, p1, c0, c1, x, wl, bl, wr, o_ref):
    cnt = jnp.maximum(c0[...] + c1[...], 1.0)
    mean = (p0[...] + p1[...]) / cnt
    y = (jnp.dot(mean, wl[...], preferred_element_type=jnp.float32)
         + bl[...]
         + jnp.dot(x[...], wr[...], preferred_element_type=jnp.float32))
    o_ref[...] = jnp.maximum(y, 0.0) if relu else y


_BLK = 400


def _dense(p0, p1, c0, c1, x, wl, bl, wr, relu):
    grid = (N_NODES // _BLK,)
    row_spec = pl.BlockSpec((_BLK, D), lambda i: (i, 0))
    cnt_spec = pl.BlockSpec((_BLK, 1), lambda i: (i, 0))
    w_spec = pl.BlockSpec((D, D), lambda i: (0, 0))
    b_spec = pl.BlockSpec((1, D), lambda i: (0, 0))
    return pl.pallas_call(
        functools.partial(_dense_body, relu),
        grid=grid,
        in_specs=[row_spec, row_spec, cnt_spec, cnt_spec,
                  row_spec, w_spec, b_spec, w_spec],
        out_specs=row_spec,
        out_shape=jax.ShapeDtypeStruct((N_NODES, D), jnp.float32),
    )(p0, p1, c0, c1, x, wl, bl, wr)


def kernel(x, edge_index, W_l1, b_l1, W_r1, W_l2, b_l2, W_r2):
    src3 = edge_index[0].reshape(NW, NCHUNK, CHUNK)
    dst3 = edge_index[1].reshape(NW, NCHUNK, CHUNK)

    part1, cnt = _agg_with_counts(x, src3, dst3)
    c0 = cnt[0, :N_NODES, None]
    c1 = cnt[1, :N_NODES, None]
    h = _dense(part1[0, :N_NODES], part1[1, :N_NODES], c0, c1,
               x, W_l1, b_l1.reshape(1, D), W_r1, relu=True)

    (part2,) = _agg_no_counts(h, src3, dst3)
    out = _dense(part2[0, :N_NODES], part2[1, :N_NODES], c0, c1,
                 h, W_l2, b_l2.reshape(1, D), W_r2, relu=False)
    return out


# speedup vs baseline: 8.4560x; 8.4560x over previous
import functools

import jax
import jax.numpy as jnp
from jax import lax
from jax.experimental import pallas as pl
from jax.experimental.pallas import tpu as pltpu
from jax.experimental.pallas import tpu_sc as plsc

N_NODES = 10000
N_EDGES = 320000
D = 128

NC, NS, LANES = 2, 16, 16
NW = NC * NS
EPW = N_EDGES // NW
CHUNK = 125
NCHUNK = EPW // CHUNK
ZROWS = 128
N_PAD = 10240
RPT = N_PAD // NS


def _agg_body(with_counts, *refs):
    if with_counts:
        (feat, src3, dst3, part_out, cnt_out,
         idx_s, idx_d, rows, ones, zv, acc, cnt) = refs
    else:
        (feat, src3, dst3, part_out,
         idx_s, idx_d, rows, acc) = refs

    cid = lax.axis_index("c")
    sid = lax.axis_index("s")
    wid = cid * NS + sid

    pltpu.sync_copy(src3.at[wid], idx_s)
    pltpu.sync_copy(dst3.at[wid], idx_d)

    def zrow(i, _):
        for j in range(D // LANES):
            rows[i, pl.ds(j * LANES, LANES)] = jnp.zeros((LANES,), jnp.float32)
        return 0
    lax.fori_loop(0, ZROWS, zrow, 0)
    base = sid * RPT
    for z in range(RPT // ZROWS):
        pltpu.sync_copy(rows.at[pl.ds(0, ZROWS)],
                        acc.at[pl.ds(base + z * ZROWS, ZROWS)])

    if with_counts:
        def zvec(i, _):
            zv[pl.ds(i * LANES, LANES)] = jnp.zeros((LANES,), jnp.float32)
            return 0
        lax.fori_loop(0, RPT // LANES, zvec, 0)
        pltpu.sync_copy(zv, cnt.at[pl.ds(base, RPT)])
        for j in range(128 // LANES):
            ones[pl.ds(j * LANES, LANES)] = jnp.ones((LANES,), jnp.float32)

    plsc.subcore_barrier()

    def body(g, _):
        buf = rows.at[pl.ds(0, CHUNK)]
        pltpu.sync_copy(feat.at[idx_s.at[g]], buf)
        pltpu.sync_copy(buf, acc.at[idx_d.at[g]], add=True)
        if with_counts:
            pltpu.sync_copy(ones.at[pl.ds(0, CHUNK)], cnt.at[idx_d.at[g]],
                            add=True)
        return 0
    lax.fori_loop(0, NCHUNK, body, 0)

    plsc.subcore_barrier()

    pltpu.sync_copy(acc.at[pl.ds(base, RPT)], part_out.at[cid, pl.ds(base, RPT)])
    if with_counts:
        pltpu.sync_copy(cnt.at[pl.ds(base, RPT)], cnt_out.at[cid, pl.ds(base, RPT)])


def _make_agg(with_counts):
    out_type = [jax.ShapeDtypeStruct((NC, N_PAD, D), jnp.float32)]
    scratch = [
        pltpu.VMEM((NCHUNK, CHUNK), jnp.int32),
        pltpu.VMEM((NCHUNK, CHUNK), jnp.int32),
        pltpu.VMEM((ZROWS, D), jnp.float32),
    ]
    if with_counts:
        out_type.append(jax.ShapeDtypeStruct((NC, N_PAD), jnp.float32))
        scratch += [
            pltpu.VMEM((128,), jnp.float32),
            pltpu.VMEM((RPT,), jnp.float32),
        ]
    scratch.append(pltpu.VMEM_SHARED((N_PAD, D), jnp.float32))
    if with_counts:
        scratch.append(pltpu.VMEM_SHARED((N_PAD,), jnp.float32))
    mesh = plsc.VectorSubcoreMesh(
        core_axis_name="c", subcore_axis_name="s",
        num_cores=NC, num_subcores=NS)
    return pl.kernel(
        functools.partial(_agg_body, with_counts),
        out_type=out_type, mesh=mesh, scratch_types=scratch)


_agg_with_counts = _make_agg(True)
_agg_no_counts = _make_agg(False)


def _dense_body(relu, p0, p1, c0, c1, x, wl, bl, wr, o_ref):
    cnt = jnp.maximum(c0[...] + c1[...], 1.0)
    mean = (p0[...] + p1[...]) / cnt
    y = (jnp.dot(mean, wl[...], preferred_element_type=jnp.float32)
         + bl[...]
         + jnp.dot(x[...], wr[...], preferred_element_type=jnp.float32))
    o_ref[...] = jnp.maximum(y, 0.0) if relu else y


_BLK = 400


def _dense(p0, p1, c0, c1, x, wl, bl, wr, relu):
    grid = (N_NODES // _BLK,)
    row_spec = pl.BlockSpec((_BLK, D), lambda i: (i, 0))
    cnt_spec = pl.BlockSpec((_BLK, 1), lambda i: (i, 0))
    w_spec = pl.BlockSpec((D, D), lambda i: (0, 0))
    b_spec = pl.BlockSpec((1, D), lambda i: (0, 0))
    return pl.pallas_call(
        functools.partial(_dense_body, relu),
        grid=grid,
        in_specs=[row_spec, row_spec, cnt_spec, cnt_spec,
                  row_spec, w_spec, b_spec, w_spec],
        out_specs=row_spec,
        out_shape=jax.ShapeDtypeStruct((N_NODES, D), jnp.float32),
    )(p0, p1, c0, c1, x, wl, bl, wr)


def kernel(x, edge_index, W_l1, b_l1, W_r1, W_l2, b_l2, W_r2):
    src3 = edge_index[0].reshape(NW, NCHUNK, CHUNK)
    dst3 = edge_index[1].reshape(NW, NCHUNK, CHUNK)

    part1, cnt = _agg_with_counts(x, src3, dst3)
    c0 = cnt[0, :N_NODES, None]
    c1 = cnt[1, :N_NODES, None]
    h = _dense(part1[0, :N_NODES], part1[1, :N_NODES], c0, c1,
               x, W_l1, b_l1.reshape(1, D), W_r1, relu=True)

    (part2,) = _agg_no_counts(h, src3, dst3)
    out = _dense(part2[0, :N_NODES], part2[1, :N_NODES], c0, c1,
                 h, W_l2, b_l2.reshape(1, D), W_r2, relu=False)
    return out
